# baseline (speedup 1.0000x reference)
import functools

import jax
import jax.numpy as jnp
from jax import lax
from jax.experimental import pallas as pl
from jax.experimental.pallas import tpu as pltpu
from jax.experimental.pallas import tpu_sc as plsc

B = 16384
N_NUM_FEATS = 64
K_FIELD = 16
K_ID = 32
OUT_DIM = N_NUM_FEATS + 2 * K_FIELD + 3 * K_ID

_info = plsc.get_sparse_core_info()
NC, NS, L = _info.num_cores, _info.num_subcores, _info.num_lanes
NW = NC * NS
BPW = B // NW
XSPW = BPW * N_NUM_FEATS // 128


def _sc_kernel(x_h, dr_h, field_h, jockey_h, horse_h, trainer_h,
               ed_h, ef_h, ej_h, eh_h, et_h, w_h, b_h, out_h,
               x_v, dri_v, fi_v, ji_v, hi_v, ti_v, sup_v,
               big_v, acc_v, w_v, b_v, out_v, sem):
    wid = lax.axis_index("s") * NC + lax.axis_index("c")
    base = wid * BPW

    pltpu.sync_copy(dr_h.at[pl.ds(base, BPW)], dri_v)
    pltpu.sync_copy(field_h.at[pl.ds(base, BPW)], fi_v)
    pltpu.sync_copy(jockey_h.at[pl.ds(base, BPW)], ji_v)
    pltpu.sync_copy(horse_h.at[pl.ds(base, BPW)], hi_v)
    pltpu.sync_copy(trainer_h.at[pl.ds(base, BPW)], ti_v)
    pltpu.sync_copy(w_h, w_v)
    pltpu.sync_copy(b_h, b_v)
    xcp = pltpu.async_copy(x_h.at[pl.ds(wid * XSPW, XSPW)], x_v, sem)

    wc = [w_v[pl.ds(c * L, L)] for c in range(OUT_DIM // L)]

    xcp.wait()

    def x_body(g, carry):
        for rl in range(L):
            o = (rl & 1) * 64
            sr = g * (L // 2) + rl // 2
            r = g * L + rl
            t = jnp.maximum(x_v[sr, pl.ds(o, L)], 0.0) * wc[0]
            t += jnp.maximum(x_v[sr, pl.ds(o + L, L)], 0.0) * wc[1]
            t += jnp.maximum(x_v[sr, pl.ds(o + 2 * L, L)], 0.0) * wc[2]
            t += jnp.maximum(x_v[sr, pl.ds(o + 3 * L, L)], 0.0) * wc[3]
            acc_v[pl.ds(r * L, L)] = t
        return carry

    lax.fori_loop(0, BPW // L, x_body, 0)

    tables = (
        (ed_h, dri_v, 3, K_FIELD, 4),
        (ef_h, fi_v, 3, K_FIELD, 5),
        (ej_h, ji_v, 2, K_ID, 6),
        (eh_h, hi_v, 2, K_ID, 8),
        (et_h, ti_v, 2, K_ID, 10),
    )

    for tbl_h, idx_v, shift, kw, wci in tables:
        def sup_body(c, carry, idx_v=idx_v, shift=shift):
            sup_v[pl.ds(c * L, L)] = idx_v[pl.ds(c * L, L)] >> shift
            return carry

        lax.fori_loop(0, BPW // L, sup_body, 0)
        pltpu.async_copy(tbl_h.at[sup_v], big_v, sem).wait()
        mask = (1 << shift) - 1

        def grp_body(g, carry, idx_v=idx_v, kw=kw, wci=wci, mask=mask):
            row0 = g * L
            iv = idx_v[pl.ds(row0, L)]
            for rl in range(L):
                o = (iv[rl] & mask) * kw
                r = row0 + rl
                t = acc_v[pl.ds(r * L, L)]
                t += jnp.maximum(big_v[r, pl.ds(o, L)], 0.0) * wc[wci]
                if kw == K_ID:
                    t += (jnp.maximum(big_v[r, pl.ds(o + L, L)], 0.0)
                          * wc[wci + 1])
                acc_v[pl.ds(r * L, L)] = t
            return carry

        lax.fori_loop(0, BPW // L, grp_body, 0)

    bias = b_v[...]
    lane_iota = lax.iota(jnp.int32, L)
    perms = [lane_iota ^ s for s in (1, 2, 4, 8)]
    dnums = lax.GatherDimensionNumbers(
        offset_dims=(), collapsed_slice_dims=(0,), start_index_map=(0,))

    def _lane_sum(t):
        for p in perms:
            t = t + lax.gather(t, p[:, None], dnums, slice_sizes=(1,),
                               mode=lax.GatherScatterMode.PROMISE_IN_BOUNDS)
        return t

    def group_body(g, carry):
        row0 = g * L
        acc = jnp.zeros((L,), jnp.float32)
        for rl in range(L):
            t = acc_v[pl.ds((row0 + rl) * L, L)]
            acc = jnp.where(lane_iota == rl, _lane_sum(t), acc)
        z = acc + bias
        out_v[pl.ds(row0, L)] = 1.0 / (1.0 + jnp.exp(-z))
        return carry

    lax.fori_loop(0, BPW // L, group_body, 0)
    pltpu.sync_copy(out_v, out_h.at[pl.ds(base, BPW)])


@jax.jit
def _run(x, dr, field, jockey, horse, trainer,
         emb_dr_w, emb_field_w, emb_jockey_w, emb_horse_w, emb_trainer_w,
         W, b):
    w_flat = W.reshape(OUT_DIM).astype(jnp.float32)
    b16 = jnp.broadcast_to(b.reshape(1), (L,)).astype(jnp.float32)
    x128 = x.astype(jnp.float32).reshape(B * N_NUM_FEATS // 128, 128)
    ed128 = emb_dr_w.reshape(-1, 128)
    ef128 = emb_field_w.reshape(-1, 128)
    ej128 = emb_jockey_w.reshape(-1, 128)
    eh128 = emb_horse_w.reshape(-1, 128)
    et128 = emb_trainer_w.reshape(-1, 128)
    mesh = plsc.VectorSubcoreMesh(core_axis_name="c", subcore_axis_name="s")
    f = functools.partial(
        pl.kernel, _sc_kernel, mesh=mesh,
        out_type=jax.ShapeDtypeStruct((B,), jnp.float32),
        scratch_types=[
            pltpu.VMEM((XSPW, 128), jnp.float32),
            pltpu.VMEM((BPW,), jnp.int32),
            pltpu.VMEM((BPW,), jnp.int32),
            pltpu.VMEM((BPW,), jnp.int32),
            pltpu.VMEM((BPW,), jnp.int32),
            pltpu.VMEM((BPW,), jnp.int32),
            pltpu.VMEM((BPW,), jnp.int32),
            pltpu.VMEM((BPW, 128), jnp.float32),
            pltpu.VMEM((BPW * L,), jnp.float32),
            pltpu.VMEM((OUT_DIM,), jnp.float32),
            pltpu.VMEM((L,), jnp.float32),
            pltpu.VMEM((BPW,), jnp.float32),
            pltpu.SemaphoreType.DMA,
        ],
    )()
    out = f(x128,
            dr.astype(jnp.int32), field.astype(jnp.int32),
            jockey.astype(jnp.int32), horse.astype(jnp.int32),
            trainer.astype(jnp.int32),
            ed128, ef128, ej128, eh128, et128,
            w_flat, b16)
    return out.reshape(B, 1)


def kernel(x, dr, field, jockey, horse, trainer, emb_dr_w, emb_field_w,
           emb_jockey_w, emb_horse_w, emb_trainer_w, W, b):
    return _run(x, dr, field, jockey, horse, trainer, emb_dr_w, emb_field_w,
                emb_jockey_w, emb_horse_w, emb_trainer_w, W, b)


# speedup vs baseline: 2.7855x; 2.7855x over previous
import functools

import jax
import jax.numpy as jnp
from jax import lax
from jax.experimental import pallas as pl
from jax.experimental.pallas import tpu as pltpu
from jax.experimental.pallas import tpu_sc as plsc

B = 16384
N_NUM_FEATS = 64
K_FIELD = 16
K_ID = 32
OUT_DIM = N_NUM_FEATS + 2 * K_FIELD + 3 * K_ID
N_DR = 1000
N_FIELD = 1000
N_JOCKEY = 100000
N_HORSE = 1000000
N_TRAINER = 100000

_info = plsc.get_sparse_core_info()
NC, NS, L = _info.num_cores, _info.num_subcores, _info.num_lanes
NW = NC * NS
BPW = B // NW


def _score_body(t_ref, w_ref, o_ref):
    o_ref[...] = jnp.sum(jnp.maximum(t_ref[...], 0.0) * w_ref[...], axis=0,
                         keepdims=True)


def _scores(tt, wseg, bn):
    k, n = tt.shape
    grid = (n + bn - 1) // bn
    out = pl.pallas_call(
        _score_body,
        grid=(grid,),
        in_specs=[pl.BlockSpec((k, bn), lambda i: (0, i)),
                  pl.BlockSpec((k, 1), lambda i: (0, 0))],
        out_specs=pl.BlockSpec((1, bn), lambda i: (0, i)),
        out_shape=jax.ShapeDtypeStruct((1, n), jnp.float32),
    )(tt, wseg)
    return out.reshape(n)


def _sc_kernel(sx_h, dr_h, field_h, jockey_h, horse_h, trainer_h,
               sd_h, sf_h, sj_h, sh_h, st_h, b_h, out_h,
               sx_v, dri_v, fi_v, ji_v, hi_v, ti_v,
               gd_v, gf_v, gj_v, gh_v, gt_v, b_v, out_v, sem):
    wid = lax.axis_index("s") * NC + lax.axis_index("c")
    base = wid * BPW

    pltpu.sync_copy(dr_h.at[pl.ds(base, BPW)], dri_v)
    pltpu.sync_copy(field_h.at[pl.ds(base, BPW)], fi_v)
    pltpu.sync_copy(jockey_h.at[pl.ds(base, BPW)], ji_v)
    pltpu.sync_copy(horse_h.at[pl.ds(base, BPW)], hi_v)
    pltpu.sync_copy(trainer_h.at[pl.ds(base, BPW)], ti_v)
    pltpu.sync_copy(b_h, b_v)

    cps = [
        pltpu.async_copy(sx_h.at[pl.ds(base, BPW)], sx_v, sem),
        pltpu.async_copy(sd_h.at[dri_v], gd_v, sem),
        pltpu.async_copy(sf_h.at[fi_v], gf_v, sem),
        pltpu.async_copy(sj_h.at[ji_v], gj_v, sem),
        pltpu.async_copy(sh_h.at[hi_v], gh_v, sem),
        pltpu.async_copy(st_h.at[ti_v], gt_v, sem),
    ]
    for cp in cps:
        cp.wait()

    bias = b_v[...]

    def body(c, carry):
        sl = pl.ds(c * L, L)
        z = (sx_v[sl] + gd_v[sl] + gf_v[sl] + gj_v[sl] + gh_v[sl] + gt_v[sl]
             + bias)
        out_v[sl] = 1.0 / (1.0 + jnp.exp(-z))
        return carry

    lax.fori_loop(0, BPW // L, body, 0)
    pltpu.sync_copy(out_v, out_h.at[pl.ds(base, BPW)])


@jax.jit
def _run(x, dr, field, jockey, horse, trainer,
         emb_dr_w, emb_field_w, emb_jockey_w, emb_horse_w, emb_trainer_w,
         W, b):
    w = W.reshape(OUT_DIM).astype(jnp.float32)
    wx = w[0:64].reshape(N_NUM_FEATS, 1)
    wd = w[64:80].reshape(K_FIELD, 1)
    wf = w[80:96].reshape(K_FIELD, 1)
    wj = w[96:128].reshape(K_ID, 1)
    wh = w[128:160].reshape(K_ID, 1)
    wt = w[160:192].reshape(K_ID, 1)

    sx = _scores(x.astype(jnp.float32).T, wx, 2048)
    sd = _scores(emb_dr_w.T, wd, 1024)
    sf = _scores(emb_field_w.T, wf, 1024)
    sj = _scores(emb_jockey_w.T, wj, 8192)
    sh = _scores(emb_horse_w.T, wh, 8192)
    st = _scores(emb_trainer_w.T, wt, 8192)

    b16 = jnp.broadcast_to(b.reshape(1), (L,)).astype(jnp.float32)
    mesh = plsc.VectorSubcoreMesh(core_axis_name="c", subcore_axis_name="s")
    f = functools.partial(
        pl.kernel, _sc_kernel, mesh=mesh,
        out_type=jax.ShapeDtypeStruct((B,), jnp.float32),
        scratch_types=[
            pltpu.VMEM((BPW,), jnp.float32),
            pltpu.VMEM((BPW,), jnp.int32),
            pltpu.VMEM((BPW,), jnp.int32),
            pltpu.VMEM((BPW,), jnp.int32),
            pltpu.VMEM((BPW,), jnp.int32),
            pltpu.VMEM((BPW,), jnp.int32),
            pltpu.VMEM((BPW,), jnp.float32),
            pltpu.VMEM((BPW,), jnp.float32),
            pltpu.VMEM((BPW,), jnp.float32),
            pltpu.VMEM((BPW,), jnp.float32),
            pltpu.VMEM((BPW,), jnp.float32),
            pltpu.VMEM((L,), jnp.float32),
            pltpu.VMEM((BPW,), jnp.float32),
            pltpu.SemaphoreType.DMA,
        ],
    )()
    out = f(sx,
            dr.astype(jnp.int32), field.astype(jnp.int32),
            jockey.astype(jnp.int32), horse.astype(jnp.int32),
            trainer.astype(jnp.int32),
            sd, sf, sj, sh, st, b16)
    return out.reshape(B, 1)


def kernel(x, dr, field, jockey, horse, trainer, emb_dr_w, emb_field_w,
           emb_jockey_w, emb_horse_w, emb_trainer_w, W, b):
    return _run(x, dr, field, jockey, horse, trainer, emb_dr_w, emb_field_w,
                emb_jockey_w, emb_horse_w, emb_trainer_w, W, b)


# speedup vs baseline: 4.5437x; 1.6312x over previous
import functools

import jax
import jax.numpy as jnp
from jax import lax
from jax.experimental import pallas as pl
from jax.experimental.pallas import tpu as pltpu
from jax.experimental.pallas import tpu_sc as plsc

B = 16384
N_NUM_FEATS = 64
K_FIELD = 16
K_ID = 32
OUT_DIM = N_NUM_FEATS + 2 * K_FIELD + 3 * K_ID
N_DR = 1000
N_FIELD = 1000
N_JOCKEY = 100000
N_HORSE = 1000000
N_TRAINER = 100000

_info = plsc.get_sparse_core_info()
NC, NS, L = _info.num_cores, _info.num_subcores, _info.num_lanes
NW = NC * NS
BPW = B // NW


def _score_body(t_ref, w_ref, o_ref):
    o_ref[...] = jnp.sum(jnp.maximum(t_ref[...], 0.0) * w_ref[...], axis=0)


def _scores(tt, wseg, bn):
    k, n = tt.shape
    grid = (n + bn - 1) // bn
    return pl.pallas_call(
        _score_body,
        grid=(grid,),
        in_specs=[pl.BlockSpec((k, bn), lambda i: (0, i)),
                  pl.BlockSpec((k, 1), lambda i: (0, 0))],
        out_specs=pl.BlockSpec((bn,), lambda i: (i,)),
        out_shape=jax.ShapeDtypeStruct((n,), jnp.float32),
    )(tt, wseg)


def _sc_kernel(sx_h, dr_h, field_h, jockey_h, horse_h, trainer_h,
               sd_h, sf_h, sj_h, sh_h, st_h, b_h, out_h,
               sx_v, dri_v, fi_v, ji_v, hi_v, ti_v,
               gd_v, gf_v, gj_v, gh_v, gt_v, b_v, out_v, sem):
    wid = lax.axis_index("s") * NC + lax.axis_index("c")
    base = wid * BPW

    pltpu.sync_copy(dr_h.at[pl.ds(base, BPW)], dri_v)
    pltpu.sync_copy(field_h.at[pl.ds(base, BPW)], fi_v)
    pltpu.sync_copy(jockey_h.at[pl.ds(base, BPW)], ji_v)
    pltpu.sync_copy(horse_h.at[pl.ds(base, BPW)], hi_v)
    pltpu.sync_copy(trainer_h.at[pl.ds(base, BPW)], ti_v)
    pltpu.sync_copy(b_h, b_v)

    cps = [
        pltpu.async_copy(sx_h.at[pl.ds(base, BPW)], sx_v, sem),
        pltpu.async_copy(sd_h.at[dri_v], gd_v, sem),
        pltpu.async_copy(sf_h.at[fi_v], gf_v, sem),
        pltpu.async_copy(sj_h.at[ji_v], gj_v, sem),
        pltpu.async_copy(sh_h.at[hi_v], gh_v, sem),
        pltpu.async_copy(st_h.at[ti_v], gt_v, sem),
    ]
    for cp in cps:
        cp.wait()

    bias = b_v[...]

    def body(c, carry):
        sl = pl.ds(c * L, L)
        z = (sx_v[sl] + gd_v[sl] + gf_v[sl] + gj_v[sl] + gh_v[sl] + gt_v[sl]
             + bias)
        out_v[sl] = 1.0 / (1.0 + jnp.exp(-z))
        return carry

    lax.fori_loop(0, BPW // L, body, 0)
    pltpu.sync_copy(out_v, out_h.at[pl.ds(base, BPW)])


@jax.jit
def _run(x, dr, field, jockey, horse, trainer,
         emb_dr_w, emb_field_w, emb_jockey_w, emb_horse_w, emb_trainer_w,
         W, b):
    w = W.reshape(OUT_DIM).astype(jnp.float32)
    wx = w[0:64].reshape(N_NUM_FEATS, 1)
    wd = w[64:80].reshape(K_FIELD, 1)
    wf = w[80:96].reshape(K_FIELD, 1)
    wj = w[96:128].reshape(K_ID, 1)
    wh = w[128:160].reshape(K_ID, 1)
    wt = w[160:192].reshape(K_ID, 1)

    sx = _scores(x.astype(jnp.float32).T, wx, 16384)
    sd = _scores(emb_dr_w.T, wd, 1024)
    sf = _scores(emb_field_w.T, wf, 1024)
    sj = _scores(emb_jockey_w.T, wj, 16384)
    sh = _scores(emb_horse_w.T, wh, 16384)
    st = _scores(emb_trainer_w.T, wt, 16384)

    b16 = jnp.broadcast_to(b.reshape(1), (L,)).astype(jnp.float32)
    mesh = plsc.VectorSubcoreMesh(core_axis_name="c", subcore_axis_name="s")
    f = functools.partial(
        pl.kernel, _sc_kernel, mesh=mesh,
        out_type=jax.ShapeDtypeStruct((B,), jnp.float32),
        scratch_types=[
            pltpu.VMEM((BPW,), jnp.float32),
            pltpu.VMEM((BPW,), jnp.int32),
            pltpu.VMEM((BPW,), jnp.int32),
            pltpu.VMEM((BPW,), jnp.int32),
            pltpu.VMEM((BPW,), jnp.int32),
            pltpu.VMEM((BPW,), jnp.int32),
            pltpu.VMEM((BPW,), jnp.float32),
            pltpu.VMEM((BPW,), jnp.float32),
            pltpu.VMEM((BPW,), jnp.float32),
            pltpu.VMEM((BPW,), jnp.float32),
            pltpu.VMEM((BPW,), jnp.float32),
            pltpu.VMEM((L,), jnp.float32),
            pltpu.VMEM((BPW,), jnp.float32),
            pltpu.SemaphoreType.DMA,
        ],
    )()
    out = f(sx,
            dr.astype(jnp.int32), field.astype(jnp.int32),
            jockey.astype(jnp.int32), horse.astype(jnp.int32),
            trainer.astype(jnp.int32),
            sd, sf, sj, sh, st, b16)
    return out.reshape(B, 1)


def kernel(x, dr, field, jockey, horse, trainer, emb_dr_w, emb_field_w,
           emb_jockey_w, emb_horse_w, emb_trainer_w, W, b):
    return _run(x, dr, field, jockey, horse, trainer, emb_dr_w, emb_field_w,
                emb_jockey_w, emb_horse_w, emb_trainer_w, W, b)


# speedup vs baseline: 5.0675x; 1.1153x over previous
import functools

import jax
import jax.numpy as jnp
from jax import lax
from jax.experimental import pallas as pl
from jax.experimental.pallas import tpu as pltpu
from jax.experimental.pallas import tpu_sc as plsc

B = 16384
N_NUM_FEATS = 64
K_FIELD = 16
K_ID = 32
OUT_DIM = N_NUM_FEATS + 2 * K_FIELD + 3 * K_ID
N_DR = 1000
N_FIELD = 1000
N_JOCKEY = 100000
N_HORSE = 1000000
N_TRAINER = 100000

_info = plsc.get_sparse_core_info()
NC, NS, L = _info.num_cores, _info.num_subcores, _info.num_lanes
NW = NC * NS
BPW = B // NW


def _score_body(t_ref, w_ref, o_ref):
    o_ref[...] = jnp.sum(jnp.maximum(t_ref[...], 0.0) * w_ref[...], axis=0)


def _scores(tt, wseg, bn):
    k, n = tt.shape
    grid = (n + bn - 1) // bn
    return pl.pallas_call(
        _score_body,
        grid=(grid,),
        in_specs=[pl.BlockSpec((k, bn), lambda i: (0, i)),
                  pl.BlockSpec((k, 1), lambda i: (0, 0))],
        out_specs=pl.BlockSpec((bn,), lambda i: (i,)),
        out_shape=jax.ShapeDtypeStruct((n,), jnp.float32),
    )(tt, wseg)


def _sc_kernel(sx_h, dr_h, field_h, jockey_h, horse_h, trainer_h,
               sd_h, sf_h, sj_h, sh_h, st_h, b_h, out_h,
               sx_v, dri_v, fi_v, ji_v, hi_v, ti_v,
               gd_v, gf_v, gj_v, gh_v, gt_v, b_v, out_v, sem):
    wid = lax.axis_index("s") * NC + lax.axis_index("c")
    base = wid * BPW

    pltpu.sync_copy(dr_h.at[pl.ds(base, BPW)], dri_v)
    pltpu.sync_copy(field_h.at[pl.ds(base, BPW)], fi_v)
    pltpu.sync_copy(jockey_h.at[pl.ds(base, BPW)], ji_v)
    pltpu.sync_copy(horse_h.at[pl.ds(base, BPW)], hi_v)
    pltpu.sync_copy(trainer_h.at[pl.ds(base, BPW)], ti_v)
    pltpu.sync_copy(b_h, b_v)

    cps = [
        pltpu.async_copy(sx_h.at[pl.ds(base, BPW)], sx_v, sem),
        pltpu.async_copy(sd_h.at[dri_v], gd_v, sem),
        pltpu.async_copy(sf_h.at[fi_v], gf_v, sem),
        pltpu.async_copy(sj_h.at[ji_v], gj_v, sem),
        pltpu.async_copy(sh_h.at[hi_v], gh_v, sem),
        pltpu.async_copy(st_h.at[ti_v], gt_v, sem),
    ]
    for cp in cps:
        cp.wait()

    bias = b_v[...]

    def body(c, carry):
        sl = pl.ds(c * L, L)
        z = (sx_v[sl] + gd_v[sl] + gf_v[sl] + gj_v[sl] + gh_v[sl] + gt_v[sl]
             + bias)
        out_v[sl] = 1.0 / (1.0 + jnp.exp(-z))
        return carry

    lax.fori_loop(0, BPW // L, body, 0)
    pltpu.sync_copy(out_v, out_h.at[pl.ds(base, BPW)])


@jax.jit
def _run(x, dr, field, jockey, horse, trainer,
         emb_dr_w, emb_field_w, emb_jockey_w, emb_horse_w, emb_trainer_w,
         W, b):
    w = W.reshape(OUT_DIM).astype(jnp.float32)
    wx = w[0:64].reshape(N_NUM_FEATS, 1)
    wd = w[64:80].reshape(K_FIELD, 1)
    wf = w[80:96].reshape(K_FIELD, 1)
    wj = w[96:128].reshape(K_ID, 1)
    wh = w[128:160].reshape(K_ID, 1)
    wt = w[160:192].reshape(K_ID, 1)

    sx = _scores(x.astype(jnp.float32).T, wx, 16384)
    sd = _scores(emb_dr_w.T, wd, 1024)
    sf = _scores(emb_field_w.T, wf, 1024)
    sj = _scores(emb_jockey_w.T, wj, 32768)
    sh = _scores(emb_horse_w.T, wh, 32768)
    st = _scores(emb_trainer_w.T, wt, 32768)

    b16 = jnp.broadcast_to(b.reshape(1), (L,)).astype(jnp.float32)
    mesh = plsc.VectorSubcoreMesh(core_axis_name="c", subcore_axis_name="s")
    f = functools.partial(
        pl.kernel, _sc_kernel, mesh=mesh,
        out_type=jax.ShapeDtypeStruct((B,), jnp.float32),
        scratch_types=[
            pltpu.VMEM((BPW,), jnp.float32),
            pltpu.VMEM((BPW,), jnp.int32),
            pltpu.VMEM((BPW,), jnp.int32),
            pltpu.VMEM((BPW,), jnp.int32),
            pltpu.VMEM((BPW,), jnp.int32),
            pltpu.VMEM((BPW,), jnp.int32),
            pltpu.VMEM((BPW,), jnp.float32),
            pltpu.VMEM((BPW,), jnp.float32),
            pltpu.VMEM((BPW,), jnp.float32),
            pltpu.VMEM((BPW,), jnp.float32),
            pltpu.VMEM((BPW,), jnp.float32),
            pltpu.VMEM((L,), jnp.float32),
            pltpu.VMEM((BPW,), jnp.float32),
            pltpu.SemaphoreType.DMA,
        ],
    )()
    out = f(sx,
            dr.astype(jnp.int32), field.astype(jnp.int32),
            jockey.astype(jnp.int32), horse.astype(jnp.int32),
            trainer.astype(jnp.int32),
            sd, sf, sj, sh, st, b16)
    return out.reshape(B, 1)


def kernel(x, dr, field, jockey, horse, trainer, emb_dr_w, emb_field_w,
           emb_jockey_w, emb_horse_w, emb_trainer_w, W, b):
    return _run(x, dr, field, jockey, horse, trainer, emb_dr_w, emb_field_w,
                emb_jockey_w, emb_horse_w, emb_trainer_w, W, b)


# speedup vs baseline: 5.4176x; 1.0691x over previous
import functools

import jax
import jax.numpy as jnp
from jax import lax
from jax.experimental import pallas as pl
from jax.experimental.pallas import tpu as pltpu
from jax.experimental.pallas import tpu_sc as plsc

B = 16384
N_NUM_FEATS = 64
K_FIELD = 16
K_ID = 32
OUT_DIM = N_NUM_FEATS + 2 * K_FIELD + 3 * K_ID
N_DR = 1000
N_FIELD = 1000
N_JOCKEY = 100000
N_HORSE = 1000000
N_TRAINER = 100000

_info = plsc.get_sparse_core_info()
NC, NS, L = _info.num_cores, _info.num_subcores, _info.num_lanes
NW = NC * NS
BPW = B // NW


def _score_body(t_ref, w_ref, o_ref):
    o_ref[...] = jnp.sum(jnp.maximum(t_ref[...], 0.0) * w_ref[...], axis=0)


def _scores(tt, wseg, bn):
    k, n = tt.shape
    grid = (n + bn - 1) // bn
    return pl.pallas_call(
        _score_body,
        grid=(grid,),
        in_specs=[pl.BlockSpec((k, bn), lambda i: (0, i)),
                  pl.BlockSpec((k, 1), lambda i: (0, 0))],
        out_specs=pl.BlockSpec((bn,), lambda i: (i,)),
        out_shape=jax.ShapeDtypeStruct((n,), jnp.float32),
    )(tt, wseg)


def _sc_kernel(sx_h, dr_h, field_h, jockey_h, horse_h, trainer_h,
               sd_h, sf_h, sj_h, sh_h, st_h, b_h, out_h,
               sx_v, dri_v, fi_v, ji_v, hi_v, ti_v,
               gd_v, gf_v, gj_v, gh_v, gt_v, b_v, out_v, sem):
    wid = lax.axis_index("s") * NC + lax.axis_index("c")
    base = wid * BPW

    icps = [
        pltpu.async_copy(dr_h.at[pl.ds(base, BPW)], dri_v, sem),
        pltpu.async_copy(field_h.at[pl.ds(base, BPW)], fi_v, sem),
        pltpu.async_copy(jockey_h.at[pl.ds(base, BPW)], ji_v, sem),
        pltpu.async_copy(horse_h.at[pl.ds(base, BPW)], hi_v, sem),
        pltpu.async_copy(trainer_h.at[pl.ds(base, BPW)], ti_v, sem),
        pltpu.async_copy(b_h, b_v, sem),
        pltpu.async_copy(sx_h.at[pl.ds(base, BPW)], sx_v, sem),
    ]
    for cp in icps:
        cp.wait()
    cps = [
        pltpu.async_copy(sd_h.at[dri_v], gd_v, sem),
        pltpu.async_copy(sf_h.at[fi_v], gf_v, sem),
        pltpu.async_copy(sj_h.at[ji_v], gj_v, sem),
        pltpu.async_copy(sh_h.at[hi_v], gh_v, sem),
        pltpu.async_copy(st_h.at[ti_v], gt_v, sem),
    ]
    for cp in cps:
        cp.wait()

    bias = b_v[...]

    def body(c, carry):
        sl = pl.ds(c * L, L)
        z = (sx_v[sl] + gd_v[sl] + gf_v[sl] + gj_v[sl] + gh_v[sl] + gt_v[sl]
             + bias)
        out_v[sl] = 1.0 / (1.0 + jnp.exp(-z))
        return carry

    lax.fori_loop(0, BPW // L, body, 0)
    pltpu.sync_copy(out_v, out_h.at[pl.ds(base, BPW)])


@jax.jit
def _run(x, dr, field, jockey, horse, trainer,
         emb_dr_w, emb_field_w, emb_jockey_w, emb_horse_w, emb_trainer_w,
         W, b):
    w = W.reshape(OUT_DIM).astype(jnp.float32)
    wx = w[0:64].reshape(N_NUM_FEATS, 1)
    wd = w[64:80].reshape(K_FIELD, 1)
    wf = w[80:96].reshape(K_FIELD, 1)
    wj = w[96:128].reshape(K_ID, 1)
    wh = w[128:160].reshape(K_ID, 1)
    wt = w[160:192].reshape(K_ID, 1)

    sx = _scores(x.astype(jnp.float32).T, wx, 16384)
    sd = _scores(emb_dr_w.T, wd, 1024)
    sf = _scores(emb_field_w.T, wf, 1024)
    sj = _scores(emb_jockey_w.T, wj, 65536)
    sh = _scores(emb_horse_w.T, wh, 65536)
    st = _scores(emb_trainer_w.T, wt, 65536)

    b16 = jnp.broadcast_to(b.reshape(1), (L,)).astype(jnp.float32)
    mesh = plsc.VectorSubcoreMesh(core_axis_name="c", subcore_axis_name="s")
    f = functools.partial(
        pl.kernel, _sc_kernel, mesh=mesh,
        out_type=jax.ShapeDtypeStruct((B,), jnp.float32),
        scratch_types=[
            pltpu.VMEM((BPW,), jnp.float32),
            pltpu.VMEM((BPW,), jnp.int32),
            pltpu.VMEM((BPW,), jnp.int32),
            pltpu.VMEM((BPW,), jnp.int32),
            pltpu.VMEM((BPW,), jnp.int32),
            pltpu.VMEM((BPW,), jnp.int32),
            pltpu.VMEM((BPW,), jnp.float32),
            pltpu.VMEM((BPW,), jnp.float32),
            pltpu.VMEM((BPW,), jnp.float32),
            pltpu.VMEM((BPW,), jnp.float32),
            pltpu.VMEM((BPW,), jnp.float32),
            pltpu.VMEM((L,), jnp.float32),
            pltpu.VMEM((BPW,), jnp.float32),
            pltpu.SemaphoreType.DMA,
        ],
    )()
    out = f(sx,
            dr.astype(jnp.int32), field.astype(jnp.int32),
            jockey.astype(jnp.int32), horse.astype(jnp.int32),
            trainer.astype(jnp.int32),
            sd, sf, sj, sh, st, b16)
    return out.reshape(B, 1)


def kernel(x, dr, field, jockey, horse, trainer, emb_dr_w, emb_field_w,
           emb_jockey_w, emb_horse_w, emb_trainer_w, W, b):
    return _run(x, dr, field, jockey, horse, trainer, emb_dr_w, emb_field_w,
                emb_jockey_w, emb_horse_w, emb_trainer_w, W, b)


# speedup vs baseline: 5.8069x; 1.0718x over previous
import functools

import jax
import jax.numpy as jnp
from jax import lax
from jax.experimental import pallas as pl
from jax.experimental.pallas import tpu as pltpu
from jax.experimental.pallas import tpu_sc as plsc

B = 16384
N_NUM_FEATS = 64
K_FIELD = 16
K_ID = 32
OUT_DIM = N_NUM_FEATS + 2 * K_FIELD + 3 * K_ID
N_DR = 1000
N_FIELD = 1000
N_JOCKEY = 100000
N_HORSE = 1000000
N_TRAINER = 100000

_info = plsc.get_sparse_core_info()
NC, NS, L = _info.num_cores, _info.num_subcores, _info.num_lanes
NW = NC * NS
BPW = B // NW


def _score_body(t_ref, w_ref, o_ref):
    o_ref[...] = jnp.sum(jnp.maximum(t_ref[...], 0.0) * w_ref[...], axis=0)


def _scores(tt, w2, woff, bn):
    k, n = tt.shape
    grid = (n + bn - 1) // bn
    wblk = woff // k
    return pl.pallas_call(
        _score_body,
        grid=(grid,),
        in_specs=[pl.BlockSpec((k, bn), lambda i: (0, i)),
                  pl.BlockSpec((k, 1), lambda i: (wblk, 0))],
        out_specs=pl.BlockSpec((bn,), lambda i: (i,)),
        out_shape=jax.ShapeDtypeStruct((n,), jnp.float32),
    )(tt, w2)


def _sc_kernel(sx_h, dr_h, field_h, jockey_h, horse_h, trainer_h,
               sd_h, sf_h, sj_h, sh_h, st_h, b_h, out_h,
               sx_v, dri_v, fi_v, ji_v, hi_v, ti_v,
               gd_v, gf_v, gj_v, gh_v, gt_v, b_v, out_v, sem):
    wid = lax.axis_index("s") * NC + lax.axis_index("c")
    base = wid * BPW

    icps = [
        pltpu.async_copy(dr_h.at[pl.ds(base, BPW)], dri_v, sem),
        pltpu.async_copy(field_h.at[pl.ds(base, BPW)], fi_v, sem),
        pltpu.async_copy(jockey_h.at[pl.ds(base, BPW)], ji_v, sem),
        pltpu.async_copy(horse_h.at[pl.ds(base, BPW)], hi_v, sem),
        pltpu.async_copy(trainer_h.at[pl.ds(base, BPW)], ti_v, sem),
        pltpu.async_copy(b_h, b_v, sem),
        pltpu.async_copy(sx_h.at[pl.ds(base, BPW)], sx_v, sem),
    ]
    for cp in icps:
        cp.wait()
    cps = [
        pltpu.async_copy(sd_h.at[dri_v], gd_v, sem),
        pltpu.async_copy(sf_h.at[fi_v], gf_v, sem),
        pltpu.async_copy(sj_h.at[ji_v], gj_v, sem),
        pltpu.async_copy(sh_h.at[hi_v], gh_v, sem),
        pltpu.async_copy(st_h.at[ti_v], gt_v, sem),
    ]
    for cp in cps:
        cp.wait()

    bias = b_v[...]

    def body(c, carry):
        sl = pl.ds(c * L, L)
        z = (sx_v[sl] + gd_v[sl] + gf_v[sl] + gj_v[sl] + gh_v[sl] + gt_v[sl]
             + bias)
        out_v[sl] = 1.0 / (1.0 + jnp.exp(-z))
        return carry

    lax.fori_loop(0, BPW // L, body, 0)
    pltpu.sync_copy(out_v, out_h.at[pl.ds(base, BPW)])


@jax.jit
def _run(x, dr, field, jockey, horse, trainer,
         emb_dr_w, emb_field_w, emb_jockey_w, emb_horse_w, emb_trainer_w,
         W, b):
    w2 = W.reshape(OUT_DIM, 1).astype(jnp.float32)

    sx = _scores(x.astype(jnp.float32).T, w2, 0, 16384)
    sd = _scores(emb_dr_w.T, w2, 64, 1024)
    sf = _scores(emb_field_w.T, w2, 80, 1024)
    sj = _scores(emb_jockey_w.T, w2, 96, 65536)
    sh = _scores(emb_horse_w.T, w2, 128, 131072)
    st = _scores(emb_trainer_w.T, w2, 160, 65536)

    b16 = jnp.broadcast_to(b.reshape(1), (L,)).astype(jnp.float32)
    mesh = plsc.VectorSubcoreMesh(core_axis_name="c", subcore_axis_name="s")
    f = functools.partial(
        pl.kernel, _sc_kernel, mesh=mesh,
        out_type=jax.ShapeDtypeStruct((B,), jnp.float32),
        scratch_types=[
            pltpu.VMEM((BPW,), jnp.float32),
            pltpu.VMEM((BPW,), jnp.int32),
            pltpu.VMEM((BPW,), jnp.int32),
            pltpu.VMEM((BPW,), jnp.int32),
            pltpu.VMEM((BPW,), jnp.int32),
            pltpu.VMEM((BPW,), jnp.int32),
            pltpu.VMEM((BPW,), jnp.float32),
            pltpu.VMEM((BPW,), jnp.float32),
            pltpu.VMEM((BPW,), jnp.float32),
            pltpu.VMEM((BPW,), jnp.float32),
            pltpu.VMEM((BPW,), jnp.float32),
            pltpu.VMEM((L,), jnp.float32),
            pltpu.VMEM((BPW,), jnp.float32),
            pltpu.SemaphoreType.DMA,
        ],
    )()
    out = f(sx,
            dr.astype(jnp.int32), field.astype(jnp.int32),
            jockey.astype(jnp.int32), horse.astype(jnp.int32),
            trainer.astype(jnp.int32),
            sd, sf, sj, sh, st, b16)
    return out.reshape(B, 1)


def kernel(x, dr, field, jockey, horse, trainer, emb_dr_w, emb_field_w,
           emb_jockey_w, emb_horse_w, emb_trainer_w, W, b):
    return _run(x, dr, field, jockey, horse, trainer, emb_dr_w, emb_field_w,
                emb_jockey_w, emb_horse_w, emb_trainer_w, W, b)
